# baseline (speedup 1.0000x reference)
import functools

import jax
import jax.numpy as jnp
from jax import lax
from jax.experimental import pallas as pl
from jax.experimental.pallas import tpu as pltpu
from jax.experimental.pallas import tpu_sc as plsc

VOCAB = 100000
N_HIDDEN = 128
B = 16384
F = 26
TOT = B * F
NW = 32
CHUNK = TOT // NW
LANES = 16

VB = 12800
TC_GRID = (VOCAB + VB - 1) // VB


def _project_body(e_ref, w_ref, b_ref, o_ref):
    e = e_ref[...]
    w = w_ref[...]
    o_ref[...] = (
        lax.dot_general(w, e, (((1,), (1,)), ((), ())),
                        preferred_element_type=jnp.float32)
        + b_ref[0, 0]
    )


def _project_table(emb, W, b2d):
    return pl.pallas_call(
        _project_body,
        grid=(TC_GRID,),
        in_specs=[
            pl.BlockSpec((VB, N_HIDDEN), lambda i: (i, 0)),
            pl.BlockSpec((1, N_HIDDEN), lambda i: (0, 0)),
            pl.BlockSpec((1, 1), lambda i: (0, 0)),
        ],
        out_specs=pl.BlockSpec((1, VB), lambda i: (0, i)),
        out_shape=jax.ShapeDtypeStruct((1, VOCAB), jnp.float32),
    )(emb, W, b2d)


def _sc_gather_body(v_hbm, idx_hbm, out_hbm, v_v, idx_v, out_v, sem_v, sem_i):
    wid = lax.axis_index("s") * 2 + lax.axis_index("c")
    base = wid * CHUNK
    cp_v = pltpu.async_copy(v_hbm, v_v, sem_v)
    cp_i = pltpu.async_copy(idx_hbm.at[pl.ds(base, CHUNK)], idx_v, sem_i)
    cp_v.wait()
    cp_i.wait()

    @plsc.parallel_loop(0, CHUNK, step=LANES, unroll=8)
    def body(i):
        idx = idx_v[pl.ds(i, LANES)]
        out_v[pl.ds(i, LANES)] = plsc.load_gather(v_v, [idx])

    pltpu.sync_copy(out_v, out_hbm.at[pl.ds(base, CHUNK)])


@functools.cache
def _sc_gather():
    mesh = plsc.VectorSubcoreMesh(core_axis_name="c", subcore_axis_name="s")
    return pl.kernel(
        _sc_gather_body,
        out_type=jax.ShapeDtypeStruct((TOT,), jnp.float32),
        mesh=mesh,
        scratch_types=[
            pltpu.VMEM((VOCAB,), jnp.float32),
            pltpu.VMEM((CHUNK,), jnp.int32),
            pltpu.VMEM((CHUNK,), jnp.float32),
            pltpu.SemaphoreType.DMA,
            pltpu.SemaphoreType.DMA,
        ],
        compiler_params=pltpu.CompilerParams(
            needs_layout_passes=False, skip_device_barrier=True
        ),
    )


def kernel(x, emb, W, b):
    v = _project_table(emb, W, b.reshape(1, 1)).reshape(VOCAB)
    idx = x.reshape(TOT).astype(jnp.int32)
    out = _sc_gather()(v, idx)
    return out.reshape(B, F, 1)


# speedup vs baseline: 1.1189x; 1.1189x over previous
import functools

import jax
import jax.numpy as jnp
from jax import lax
from jax.experimental import pallas as pl
from jax.experimental.pallas import tpu as pltpu
from jax.experimental.pallas import tpu_sc as plsc

VOCAB = 100000
N_HIDDEN = 128
B = 16384
F = 26
TOT = B * F
NW = 32
ROWS_W = B // NW
HALF = ROWS_W // 2
LANES = 16
XW = 32

VB = 12800
TC_GRID = (VOCAB + VB - 1) // VB
XB = B // TC_GRID


def _project_body(e_ref, w_ref, b_ref, x_ref, o_ref, xp_ref):
    e = e_ref[...]
    w = w_ref[...]
    o_ref[...] = (
        lax.dot_general(w, e, (((1,), (1,)), ((), ())),
                        preferred_element_type=jnp.float32)
        + b_ref[0, 0]
    )
    x = x_ref[...]
    xp_ref[...] = jnp.concatenate(
        [x, jnp.zeros((XB, 128 - F), jnp.int32)], axis=1
    )


def _project_table(emb, W, b2d, x):
    return pl.pallas_call(
        _project_body,
        grid=(TC_GRID,),
        in_specs=[
            pl.BlockSpec((VB, N_HIDDEN), lambda i: (i, 0)),
            pl.BlockSpec((1, N_HIDDEN), lambda i: (0, 0)),
            pl.BlockSpec((1, 1), lambda i: (0, 0)),
            pl.BlockSpec((XB, F), lambda i: (i, 0)),
        ],
        out_specs=[
            pl.BlockSpec((1, VB), lambda i: (0, i)),
            pl.BlockSpec((XB, 128), lambda i: (i, 0)),
        ],
        out_shape=[
            jax.ShapeDtypeStruct((1, VOCAB), jnp.float32),
            jax.ShapeDtypeStruct((B, 128), jnp.int32),
        ],
    )(emb, W, b2d, x)


def _sc_gather_body(v_hbm, xp_hbm, yp_hbm, v_v, xin_v, out_v, sem_v, sem_i):
    wid = lax.axis_index("s") * 2 + lax.axis_index("c")
    base = wid * ROWS_W
    cp_v = pltpu.async_copy(v_hbm, v_v, sem_v)
    cp0 = pltpu.async_copy(
        xp_hbm.at[pl.ds(base, HALF), pl.ds(0, XW)], xin_v, sem_i
    )
    cp_v.wait()

    def run_half(h):
        @plsc.parallel_loop(0, HALF, step=1, unroll=4)
        def per_row(r):
            for k in range(2):
                idx = xin_v[r, pl.ds(k * LANES, LANES)]
                out_v[h * HALF + r, pl.ds(k * LANES, LANES)] = (
                    plsc.load_gather(v_v, [idx])
                )

    cp0.wait()
    run_half(0)
    cp1 = pltpu.async_copy(
        xp_hbm.at[pl.ds(base + HALF, HALF), pl.ds(0, XW)], xin_v, sem_i
    )
    cp1.wait()
    run_half(1)
    pltpu.sync_copy(out_v, yp_hbm.at[pl.ds(base, ROWS_W), pl.ds(0, XW)])


@functools.cache
def _sc_gather():
    mesh = plsc.VectorSubcoreMesh(core_axis_name="c", subcore_axis_name="s")
    return pl.kernel(
        _sc_gather_body,
        out_type=jax.ShapeDtypeStruct((B, 128), jnp.float32),
        mesh=mesh,
        scratch_types=[
            pltpu.VMEM((VOCAB,), jnp.float32),
            pltpu.VMEM((HALF, XW), jnp.int32),
            pltpu.VMEM((ROWS_W, XW), jnp.float32),
            pltpu.SemaphoreType.DMA,
            pltpu.SemaphoreType.DMA,
        ],
        compiler_params=pltpu.CompilerParams(
            needs_layout_passes=False, use_tc_tiling_on_sc=False
        ),
    )


def kernel(x, emb, W, b):
    v, xp = _project_table(emb, W, b.reshape(1, 1), x.astype(jnp.int32))
    yp = _sc_gather()(v.reshape(VOCAB), xp)
    return yp[:, :F].reshape(B, F, 1)
